# baseline (speedup 1.0000x reference)
import jax
import jax.numpy as jnp
from jax.experimental import pallas as pl
from jax.experimental.pallas import tpu as pltpu

N = 10000
IN = 256
OUT = 128
H1 = 256
TN = 5000


def _fused_encoder_kernel(x_ref, wc1_ref, bc1_ref, wc2_ref, bc2_ref, out_ref):
    x = x_ref[...]
    p = jnp.dot(x, wc1_ref[...], preferred_element_type=jnp.float32) + bc1_ref[...]
    z1 = jax.nn.sigmoid(p[:, :H1])
    t1 = jnp.tanh(p[:, H1:])
    h = jax.nn.relu((1.0 - z1) * t1).astype(jnp.bfloat16)
    q = jnp.dot(h, wc2_ref[...], preferred_element_type=jnp.float32) + bc2_ref[...]
    out_ref[...] = (1.0 - jax.nn.sigmoid(q[:, :OUT])) * jnp.tanh(q[:, OUT:])


def kernel(x, edge_index, W1z, b1z, W1r, b1r, W1h, b1h, W2z, b2z, W2r, b2r, W2h, b2h):
    wc1 = jnp.concatenate(
        [(W1z[0, 0] + W1z[1, 0])[:IN], (W1h[0, 0] + W1h[1, 0])[:IN]], axis=1
    ).astype(jnp.bfloat16)
    bc1 = jnp.concatenate([b1z, b1h])[None, :]
    wc2 = jnp.concatenate(
        [(W2z[0, 0] + W2z[1, 0])[:H1], (W2h[0, 0] + W2h[1, 0])[:H1]], axis=1
    ).astype(jnp.bfloat16)
    bc2 = jnp.concatenate([b2z, b2h])[None, :]
    xb = x.astype(jnp.bfloat16)

    return pl.pallas_call(
        _fused_encoder_kernel,
        grid=(N // TN,),
        in_specs=[
            pl.BlockSpec((TN, IN), lambda i: (i, 0)),
            pl.BlockSpec((IN, 2 * H1), lambda i: (0, 0)),
            pl.BlockSpec((1, 2 * H1), lambda i: (0, 0)),
            pl.BlockSpec((H1, 2 * OUT), lambda i: (0, 0)),
            pl.BlockSpec((1, 2 * OUT), lambda i: (0, 0)),
        ],
        out_specs=pl.BlockSpec((TN, OUT), lambda i: (i, 0)),
        out_shape=jax.ShapeDtypeStruct((N, OUT), jnp.float32),
        compiler_params=pltpu.CompilerParams(
            dimension_semantics=("arbitrary",),
        ),
    )(xb, wc1, bc1, wc2, bc2)


# speedup vs baseline: 1.2078x; 1.2078x over previous
import jax
import jax.numpy as jnp
from jax.experimental import pallas as pl
from jax.experimental.pallas import tpu as pltpu

N = 10000
IN = 256
OUT = 128
H1 = 256
TN = 5000


def _fused_encoder_kernel(x_ref, wc1_ref, bc1_ref, wc2_ref, bc2_ref, out_ref):
    x = x_ref[...].astype(jnp.bfloat16)
    p = jnp.dot(x, wc1_ref[...], preferred_element_type=jnp.float32) + bc1_ref[...]
    z1 = jax.nn.sigmoid(p[:, :H1])
    t1 = jnp.tanh(p[:, H1:])
    h = jax.nn.relu((1.0 - z1) * t1).astype(jnp.bfloat16)
    q = jnp.dot(h, wc2_ref[...], preferred_element_type=jnp.float32) + bc2_ref[...]
    out_ref[...] = (1.0 - jax.nn.sigmoid(q[:, :OUT])) * jnp.tanh(q[:, OUT:])


def kernel(x, edge_index, W1z, b1z, W1r, b1r, W1h, b1h, W2z, b2z, W2r, b2r, W2h, b2h):
    wc1 = jnp.concatenate(
        [(W1z[0, 0] + W1z[1, 0])[:IN], (W1h[0, 0] + W1h[1, 0])[:IN]], axis=1
    ).astype(jnp.bfloat16)
    bc1 = jnp.concatenate([b1z, b1h])[None, :]
    wc2 = jnp.concatenate(
        [(W2z[0, 0] + W2z[1, 0])[:H1], (W2h[0, 0] + W2h[1, 0])[:H1]], axis=1
    ).astype(jnp.bfloat16)
    bc2 = jnp.concatenate([b2z, b2h])[None, :]

    return pl.pallas_call(
        _fused_encoder_kernel,
        grid=(N // TN,),
        in_specs=[
            pl.BlockSpec((TN, IN), lambda i: (i, 0)),
            pl.BlockSpec((IN, 2 * H1), lambda i: (0, 0)),
            pl.BlockSpec((1, 2 * H1), lambda i: (0, 0)),
            pl.BlockSpec((H1, 2 * OUT), lambda i: (0, 0)),
            pl.BlockSpec((1, 2 * OUT), lambda i: (0, 0)),
        ],
        out_specs=pl.BlockSpec((TN, OUT), lambda i: (i, 0)),
        out_shape=jax.ShapeDtypeStruct((N, OUT), jnp.float32),
        compiler_params=pltpu.CompilerParams(
            dimension_semantics=("arbitrary",),
        ),
    )(x, wc1, bc1, wc2, bc2)


# speedup vs baseline: 2.0039x; 1.6592x over previous
import jax
import jax.numpy as jnp
from jax.experimental import pallas as pl
from jax.experimental.pallas import tpu as pltpu

N = 10000
IN = 256
OUT = 128
H1 = 256
TN = 5000


def _fused_encoder_kernel(
    x_ref, w1z_ref, w1h_ref, w2z_ref, w2h_ref,
    b1z_ref, b1h_ref, b2z_ref, b2h_ref, out_ref,
):
    wz1 = (w1z_ref[0, 0] + w1z_ref[1, 0]).astype(jnp.bfloat16)
    wh1 = (w1h_ref[0, 0] + w1h_ref[1, 0]).astype(jnp.bfloat16)
    wz2 = (w2z_ref[0, 0] + w2z_ref[1, 0]).astype(jnp.bfloat16)
    wh2 = (w2h_ref[0, 0] + w2h_ref[1, 0]).astype(jnp.bfloat16)

    x = x_ref[...].astype(jnp.bfloat16)
    z1 = jax.nn.sigmoid(
        jnp.dot(x, wz1, preferred_element_type=jnp.float32) + b1z_ref[...]
    )
    t1 = jnp.tanh(
        jnp.dot(x, wh1, preferred_element_type=jnp.float32) + b1h_ref[...]
    )
    h = jax.nn.relu((1.0 - z1) * t1).astype(jnp.bfloat16)
    z2 = jax.nn.sigmoid(
        jnp.dot(h, wz2, preferred_element_type=jnp.float32) + b2z_ref[...]
    )
    t2 = jnp.tanh(
        jnp.dot(h, wh2, preferred_element_type=jnp.float32) + b2h_ref[...]
    )
    out_ref[...] = (1.0 - z2) * t2


def kernel(x, edge_index, W1z, b1z, W1r, b1r, W1h, b1h, W2z, b2z, W2r, b2r, W2h, b2h):
    wspec1 = pl.BlockSpec((2, 1, IN, H1), lambda i: (0, 0, 0, 0))
    wspec2 = pl.BlockSpec((2, 1, H1, OUT), lambda i: (0, 0, 0, 0))
    bspec1 = pl.BlockSpec((1, H1), lambda i: (0, 0))
    bspec2 = pl.BlockSpec((1, OUT), lambda i: (0, 0))
    return pl.pallas_call(
        _fused_encoder_kernel,
        grid=(N // TN,),
        in_specs=[
            pl.BlockSpec((TN, IN), lambda i: (i, 0)),
            wspec1, wspec1, wspec2, wspec2,
            bspec1, bspec1, bspec2, bspec2,
        ],
        out_specs=pl.BlockSpec((TN, OUT), lambda i: (i, 0)),
        out_shape=jax.ShapeDtypeStruct((N, OUT), jnp.float32),
        compiler_params=pltpu.CompilerParams(
            dimension_semantics=("arbitrary",),
        ),
    )(
        x, W1z, W1h, W2z, W2h,
        b1z[None, :], b1h[None, :], b2z[None, :], b2h[None, :],
    )


# speedup vs baseline: 2.0133x; 1.0047x over previous
import jax
import jax.numpy as jnp
from jax.experimental import pallas as pl
from jax.experimental.pallas import tpu as pltpu

N = 10000
IN = 256
OUT = 128
H1 = 256
TN = 2000


def _fused_encoder_kernel(
    x_ref, w1z_ref, w1h_ref, w2z_ref, w2h_ref,
    b1z_ref, b1h_ref, b2z_ref, b2h_ref, out_ref,
):
    wz1 = (w1z_ref[0, 0] + w1z_ref[1, 0]).astype(jnp.bfloat16)
    wh1 = (w1h_ref[0, 0] + w1h_ref[1, 0]).astype(jnp.bfloat16)
    wz2 = (w2z_ref[0, 0] + w2z_ref[1, 0]).astype(jnp.bfloat16)
    wh2 = (w2h_ref[0, 0] + w2h_ref[1, 0]).astype(jnp.bfloat16)

    x = x_ref[...].astype(jnp.bfloat16)
    z1 = jax.nn.sigmoid(
        jnp.dot(x, wz1, preferred_element_type=jnp.float32) + b1z_ref[...]
    )
    t1 = jnp.tanh(
        jnp.dot(x, wh1, preferred_element_type=jnp.float32) + b1h_ref[...]
    )
    h = jax.nn.relu((1.0 - z1) * t1).astype(jnp.bfloat16)
    z2 = jax.nn.sigmoid(
        jnp.dot(h, wz2, preferred_element_type=jnp.float32) + b2z_ref[...]
    )
    t2 = jnp.tanh(
        jnp.dot(h, wh2, preferred_element_type=jnp.float32) + b2h_ref[...]
    )
    out_ref[...] = (1.0 - z2) * t2


def kernel(x, edge_index, W1z, b1z, W1r, b1r, W1h, b1h, W2z, b2z, W2r, b2r, W2h, b2h):
    wspec1 = pl.BlockSpec((2, 1, IN, H1), lambda i: (0, 0, 0, 0))
    wspec2 = pl.BlockSpec((2, 1, H1, OUT), lambda i: (0, 0, 0, 0))
    bspec1 = pl.BlockSpec((1, H1), lambda i: (0, 0))
    bspec2 = pl.BlockSpec((1, OUT), lambda i: (0, 0))
    return pl.pallas_call(
        _fused_encoder_kernel,
        grid=(N // TN,),
        in_specs=[
            pl.BlockSpec((TN, IN), lambda i: (i, 0)),
            wspec1, wspec1, wspec2, wspec2,
            bspec1, bspec1, bspec2, bspec2,
        ],
        out_specs=pl.BlockSpec((TN, OUT), lambda i: (i, 0)),
        out_shape=jax.ShapeDtypeStruct((N, OUT), jnp.float32),
        compiler_params=pltpu.CompilerParams(
            dimension_semantics=("arbitrary",),
        ),
    )(
        x, W1z, W1h, W2z, W2h,
        b1z[None, :], b1h[None, :], b2z[None, :], b2h[None, :],
    )
